# Initial kernel scaffold; baseline (speedup 1.0000x reference)
import jax
import jax.numpy as jnp
from jax.experimental import pallas as pl


def kernel(values, buffer):
    raise NotImplementedError("write your pallas kernel here")


# speedup vs baseline: 29.1202x; 29.1202x over previous
import functools

import jax
import jax.numpy as jnp
from jax import lax
from jax.experimental import pallas as pl
from jax.experimental.pallas import tpu as pltpu
from jax.experimental.pallas import tpu_sc as plsc

_MAX_LEN = 4194304
_N_APPENDS = 6291456
_H = _N_APPENDS - _MAX_LEN
_NW = 32
_PER_W = _MAX_LEN // _NW


@functools.partial(
    pl.kernel,
    mesh=plsc.VectorSubcoreMesh(core_axis_name="c", subcore_axis_name="s"),
    out_type=jax.ShapeDtypeStruct((_MAX_LEN,), jnp.float32),
)
def _ring_rotate(values_hbm, out_hbm):
    wid = lax.axis_index("s") * 2 + lax.axis_index("c")
    dst = wid * _PER_W
    src = dst + jnp.where(dst < _H, _MAX_LEN, 0)
    pltpu.sync_copy(
        values_hbm.at[pl.ds(src, _PER_W)],
        out_hbm.at[pl.ds(dst, _PER_W)],
    )


def kernel(values, buffer):
    del buffer
    return _ring_rotate(values)
